# baseline (speedup 1.0000x reference)
import functools

import jax
import jax.numpy as jnp
from jax import lax
from jax.experimental import pallas as pl
from jax.experimental.pallas import tpu as pltpu
from jax.experimental.pallas import tpu_sc as plsc

_B, _T, _D = 4, 4096, 1024
_NW = 32
_T_PER_W = _T // _NW
_CHUNK = 16
_N_CHUNKS = _T_PER_W // _CHUNK
_CE = _CHUNK * _D
_N_VECS = _CE // 16
_N_STEPS = _N_CHUNKS * _B


def _sc_body(x_hbm, pe_hbm, out_hbm,
             x0, x1, p0, p1,
             sx0, sx1, sp0, sp1, ss0, ss1):
    wid = lax.axis_index("s") * 2 + lax.axis_index("c")
    t_base = wid * _T_PER_W * _D

    xbufs, pbufs = (x0, x1), (p0, p1)
    sxs, sps, sss = (sx0, sx1), (sp0, sp1), (ss0, ss1)

    def pe_off(ci):
        return t_base + ci * _CE

    def x_off(s):
        ci, b = divmod(s, _B)
        return b * (_T * _D) + pe_off(ci)

    def load_x(s):
        h = pltpu.make_async_copy(
            x_hbm.at[pl.ds(x_off(s), _CE)], xbufs[s % 2], sxs[s % 2])
        h.start()
        return h

    def load_pe(ci):
        h = pltpu.make_async_copy(
            pe_hbm.at[pl.ds(pe_off(ci), _CE)], pbufs[ci % 2], sps[ci % 2])
        h.start()
        return h

    def store_out(s):
        h = pltpu.make_async_copy(
            xbufs[s % 2], out_hbm.at[pl.ds(x_off(s), _CE)], sss[s % 2])
        h.start()
        return h

    pe_h = [None, None]
    pe_h[0] = load_pe(0)
    x_h = load_x(0)
    st_h = [None, None]

    for s in range(_N_STEPS):
        ci, b = divmod(s, _B)
        xb = xbufs[s % 2]
        pb = pbufs[ci % 2]
        if b == 0:
            pe_h[ci % 2].wait()
        x_h.wait()
        if s + 1 < _N_STEPS:
            if st_h[(s + 1) % 2] is not None:
                st_h[(s + 1) % 2].wait()
            x_h = load_x(s + 1)
            if b == _B - 1:
                pe_h[(ci + 1) % 2] = load_pe(ci + 1)

        def add_block(k, _):
            base = k * 256
            for u in range(16):
                sl = pl.ds(base + u * 16, 16)
                xb[sl] = xb[sl] + pb[sl]
            return 0

        lax.fori_loop(0, _N_VECS // 16, add_block, 0)

        st_h[s % 2] = store_out(s)

    st_h[0].wait()
    st_h[1].wait()


def kernel(x, pe):
    b, t, d = x.shape
    x_flat = x.reshape(b * t * d)
    pe_flat = pe.reshape(pe.shape[0] * pe.shape[1])
    mesh = plsc.VectorSubcoreMesh(core_axis_name="c", subcore_axis_name="s")
    sc_add = functools.partial(
        pl.kernel,
        mesh=mesh,
        out_type=jax.ShapeDtypeStruct((b * t * d,), jnp.float32),
        scratch_types=[
            pltpu.VMEM((_CE,), jnp.float32),
            pltpu.VMEM((_CE,), jnp.float32),
            pltpu.VMEM((_CE,), jnp.float32),
            pltpu.VMEM((_CE,), jnp.float32),
            pltpu.SemaphoreType.DMA,
            pltpu.SemaphoreType.DMA,
            pltpu.SemaphoreType.DMA,
            pltpu.SemaphoreType.DMA,
            pltpu.SemaphoreType.DMA,
            pltpu.SemaphoreType.DMA,
        ],
    )(_sc_body)
    out_flat = sc_add(x_flat, pe_flat)
    return out_flat.reshape(b, t, d)


# speedup vs baseline: 4.9298x; 4.9298x over previous
import jax
import jax.numpy as jnp
from jax.experimental import pallas as pl

_T_TILE = 512


def _add_pe_kernel(x_ref, pe_ref, o_ref):
    o_ref[...] = x_ref[...] + pe_ref[...]


def kernel(x, pe):
    b, t, d = x.shape
    t_tiles = t // _T_TILE
    return pl.pallas_call(
        _add_pe_kernel,
        grid=(t_tiles,),
        in_specs=[
            pl.BlockSpec((b, _T_TILE, d), lambda tt: (0, tt, 0)),
            pl.BlockSpec((1, _T_TILE, d), lambda tt: (0, tt, 0)),
        ],
        out_specs=pl.BlockSpec((b, _T_TILE, d), lambda tt: (0, tt, 0)),
        out_shape=jax.ShapeDtypeStruct((b, t, d), x.dtype),
    )(x, pe[None])


# speedup vs baseline: 4.9728x; 1.0087x over previous
import jax
import jax.numpy as jnp
from jax.experimental import pallas as pl

_T_TILE = 1024
_B_TILE = 2


def _add_pe_kernel(x_ref, pe_ref, o_ref):
    o_ref[...] = x_ref[...] + pe_ref[...]


def kernel(x, pe):
    b, t, d = x.shape
    grid = (t // _T_TILE, b // _B_TILE)
    return pl.pallas_call(
        _add_pe_kernel,
        grid=grid,
        in_specs=[
            pl.BlockSpec((_B_TILE, _T_TILE, d), lambda tt, bb: (bb, tt, 0)),
            pl.BlockSpec((1, _T_TILE, d), lambda tt, bb: (0, tt, 0)),
        ],
        out_specs=pl.BlockSpec((_B_TILE, _T_TILE, d), lambda tt, bb: (bb, tt, 0)),
        out_shape=jax.ShapeDtypeStruct((b, t, d), x.dtype),
    )(x, pe[None])


# speedup vs baseline: 5.0280x; 1.0111x over previous
import jax
import jax.numpy as jnp
from jax.experimental import pallas as pl

_T_TILE = 2048


def _add_pe_kernel(x_ref, pe_ref, o_ref):
    o_ref[0] = x_ref[0] + pe_ref[...]


def kernel(x, pe):
    b, t, d = x.shape
    grid = (t // _T_TILE, b)
    return pl.pallas_call(
        _add_pe_kernel,
        grid=grid,
        in_specs=[
            pl.BlockSpec((1, _T_TILE, d), lambda tt, bb: (bb, tt, 0)),
            pl.BlockSpec((_T_TILE, d), lambda tt, bb: (tt, 0)),
        ],
        out_specs=pl.BlockSpec((1, _T_TILE, d), lambda tt, bb: (bb, tt, 0)),
        out_shape=jax.ShapeDtypeStruct((b, t, d), x.dtype),
    )(x, pe)
